# baseline (speedup 1.0000x reference)
import jax
import jax.numpy as jnp
from jax import lax
from jax.experimental import pallas as pl
from jax.experimental.pallas import tpu as pltpu
from jax.experimental.pallas import tpu_sc as plsc

_EMB = 64
_B = 1024
_L = 200

_NC = 2
_NS = 16
_NW = _NC * _NS
_BPW = _B // _NW
_CH = 40
_NH = _L // _CH

_TPW = _BPW * _L


def _posseg_body(pos_ref, seg_ref, out_ref):
    seg = seg_ref[...]
    pos = pos_ref[...]
    out_ref[...] = seg[:, None, :] + pos[None, :, :]


def _posseg(pos_table, seg_table):
    out = pl.pallas_call(
        _posseg_body,
        out_shape=jax.ShapeDtypeStruct((3, _L, _EMB), jnp.float32),
    )(pos_table, seg_table)
    return out.reshape(3 * _L, _EMB)


def _sc_body(x_hbm, s_hbm, tt_hbm, ps_hbm, out_hbm,
             idx_v, psidx_v, buf_v, tok_sems, add_sems, wr_sems, seg_sem):
    w = lax.axis_index("s") * _NC + lax.axis_index("c")
    b0 = w * _BPW

    pltpu.sync_copy(x_hbm.at[pl.ds(b0, _BPW)], idx_v)
    for r in range(_BPW):
        pltpu.async_copy(s_hbm.at[b0 + r], psidx_v.at[pl.ds(r * _L, _L)],
                         seg_sem)
    for r in range(_BPW):
        pltpu.make_async_copy(s_hbm.at[b0 + r],
                              psidx_v.at[pl.ds(r * _L, _L)], seg_sem).wait()

    iota = lax.iota(jnp.int32, 16)

    def idx_body(g, carry):
        f = g * 16
        s16 = psidx_v[pl.ds(f, 16)]
        psidx_v[pl.ds(f, 16)] = s16 * _L + lax.rem(f + iota, _L)
        return carry

    lax.fori_loop(0, _TPW // 16, idx_body, 0)

    def row_body(r, carry):
        b = b0 + r
        for h in range(_NH):
            @pl.when(r > 0)
            def _():
                pltpu.make_async_copy(
                    buf_v.at[h], out_hbm.at[b - 1, pl.ds(h * _CH, _CH)],
                    wr_sems.at[h]).wait()

            pltpu.async_copy(tt_hbm.at[idx_v.at[r, pl.ds(h * _CH, _CH)]],
                             buf_v.at[h], tok_sems.at[h])
        for h in range(_NH):
            pltpu.make_async_copy(
                tt_hbm.at[idx_v.at[r, pl.ds(h * _CH, _CH)]], buf_v.at[h],
                tok_sems.at[h]).wait()
            pltpu.async_copy(ps_hbm.at[psidx_v.at[pl.ds(r * _L + h * _CH, _CH)]],
                             buf_v.at[h], add_sems.at[h], add=True)
        for h in range(_NH):
            pltpu.make_async_copy(
                ps_hbm.at[psidx_v.at[pl.ds(r * _L + h * _CH, _CH)]], buf_v.at[h],
                add_sems.at[h]).wait()
            pltpu.async_copy(buf_v.at[h], out_hbm.at[b, pl.ds(h * _CH, _CH)],
                             wr_sems.at[h])
        return carry

    lax.fori_loop(0, _BPW, row_body, 0)

    for h in range(_NH):
        pltpu.make_async_copy(
            buf_v.at[h], out_hbm.at[b0 + _BPW - 1, pl.ds(h * _CH, _CH)],
            wr_sems.at[h]).wait()


def _sc_call(x, segment_label, token_table, posseg):
    mesh = plsc.VectorSubcoreMesh(core_axis_name="c", subcore_axis_name="s")
    fn = pl.kernel(
        _sc_body,
        out_type=jax.ShapeDtypeStruct((_B, _L, _EMB), jnp.float32),
        mesh=mesh,
        compiler_params=pltpu.CompilerParams(use_tc_tiling_on_sc=False),
        scratch_types=[
            pltpu.VMEM((_BPW, _L), jnp.int32),
            pltpu.VMEM((_TPW,), jnp.int32),
            pltpu.VMEM((_NH, _CH, _EMB), jnp.float32),
            pltpu.SemaphoreType.DMA((_NH,)),
            pltpu.SemaphoreType.DMA((_NH,)),
            pltpu.SemaphoreType.DMA((_NH,)),
            pltpu.SemaphoreType.DMA,
        ],
    )
    return fn(x, segment_label, token_table, posseg)


def kernel(x, segment_label, token_table, pos_table, seg_table):
    posseg = _posseg(pos_table, seg_table)
    return _sc_call(x.astype(jnp.int32), segment_label.astype(jnp.int32),
                    token_table, posseg)


# speedup vs baseline: 1.0937x; 1.0937x over previous
import jax
import jax.numpy as jnp
from jax import lax
from jax.experimental import pallas as pl
from jax.experimental.pallas import tpu as pltpu
from jax.experimental.pallas import tpu_sc as plsc

_EMB = 64
_B = 1024
_L = 200

_NC = 2
_NS = 16
_NW = _NC * _NS
_BPW = _B // _NW
_CH = 40
_NH = _L // _CH

_TPW = _BPW * _L


def _posseg_body(pos_ref, seg_ref, out_ref):
    seg = seg_ref[...]
    pos = pos_ref[...]
    out_ref[...] = seg[:, None, :] + pos[None, :, :]


def _posseg(pos_table, seg_table):
    out = pl.pallas_call(
        _posseg_body,
        out_shape=jax.ShapeDtypeStruct((3, _L, _EMB), jnp.float32),
    )(pos_table, seg_table)
    return out.reshape(3 * _L, _EMB)


def _sc_body(x_hbm, s_hbm, tt_hbm, ps_hbm, out_hbm,
             idx_v, psidx_v, buf_v, tok_sems, add_sems, wr_sems, seg_sem):
    w = lax.axis_index("s") * _NC + lax.axis_index("c")
    b0 = w * _BPW

    pltpu.sync_copy(x_hbm.at[pl.ds(b0, _BPW)], idx_v)
    for r in range(_BPW):
        pltpu.async_copy(s_hbm.at[b0 + r], psidx_v.at[pl.ds(r * _L, _L)],
                         seg_sem)
    for r in range(_BPW):
        pltpu.make_async_copy(s_hbm.at[b0 + r],
                              psidx_v.at[pl.ds(r * _L, _L)], seg_sem).wait()

    iota = lax.iota(jnp.int32, 16)

    def idx_body(g, carry):
        f = g * 16
        s16 = psidx_v[pl.ds(f, 16)]
        psidx_v[pl.ds(f, 16)] = s16 * _L + lax.rem(f + iota, _L)
        return carry

    lax.fori_loop(0, _TPW // 16, idx_body, 0)

    def row_body(r, carry):
        b = b0 + r
        for h in range(_NH):
            @pl.when(r > 0)
            def _():
                pltpu.make_async_copy(
                    buf_v.at[h],
                    out_hbm.at[b - 1, pl.ds(h * _CH, _CH), pl.ds(0, _EMB)],
                    wr_sems.at[h]).wait()

            pltpu.async_copy(tt_hbm.at[idx_v.at[r, pl.ds(h * _CH, _CH)]],
                             buf_v.at[h], tok_sems.at[h])
        for h in range(_NH):
            pltpu.make_async_copy(
                tt_hbm.at[idx_v.at[r, pl.ds(h * _CH, _CH)]], buf_v.at[h],
                tok_sems.at[h]).wait()
            pltpu.async_copy(ps_hbm.at[psidx_v.at[pl.ds(r * _L + h * _CH, _CH)]],
                             buf_v.at[h], add_sems.at[h], add=True)
        for h in range(_NH):
            pltpu.make_async_copy(
                ps_hbm.at[psidx_v.at[pl.ds(r * _L + h * _CH, _CH)]], buf_v.at[h],
                add_sems.at[h]).wait()
            pltpu.async_copy(buf_v.at[h],
                             out_hbm.at[b, pl.ds(h * _CH, _CH), pl.ds(0, _EMB)],
                             wr_sems.at[h])
        return carry

    lax.fori_loop(0, _BPW, row_body, 0)

    for h in range(_NH):
        pltpu.make_async_copy(
            buf_v.at[h],
            out_hbm.at[b0 + _BPW - 1, pl.ds(h * _CH, _CH), pl.ds(0, _EMB)],
            wr_sems.at[h]).wait()


def _sc_call(x, segment_label, token_table, posseg):
    mesh = plsc.VectorSubcoreMesh(core_axis_name="c", subcore_axis_name="s")
    fn = pl.kernel(
        _sc_body,
        out_type=jax.ShapeDtypeStruct((_B, _L, 2 * _EMB), jnp.float32),
        mesh=mesh,
        compiler_params=pltpu.CompilerParams(use_tc_tiling_on_sc=False),
        scratch_types=[
            pltpu.VMEM((_BPW, _L), jnp.int32),
            pltpu.VMEM((_TPW,), jnp.int32),
            pltpu.VMEM((_NH, _CH, _EMB), jnp.float32),
            pltpu.SemaphoreType.DMA((_NH,)),
            pltpu.SemaphoreType.DMA((_NH,)),
            pltpu.SemaphoreType.DMA((_NH,)),
            pltpu.SemaphoreType.DMA,
        ],
    )
    return fn(x, segment_label, token_table, posseg)


def kernel(x, segment_label, token_table, pos_table, seg_table):
    posseg = _posseg(pos_table, seg_table)
    out = _sc_call(x.astype(jnp.int32), segment_label.astype(jnp.int32),
                    token_table, posseg)
    return out[:, :, :_EMB]


# speedup vs baseline: 1.5746x; 1.4397x over previous
import jax
import jax.numpy as jnp
from jax import lax
from jax.experimental import pallas as pl
from jax.experimental.pallas import tpu as pltpu
from jax.experimental.pallas import tpu_sc as plsc
from jax.experimental import layout as jex_layout

_EMB = 64
_B = 1024
_L = 200

_NC = 2
_NS = 16
_NW = _NC * _NS
_BPW = _B // _NW
_CH = 40
_NH = _L // _CH

_TPW = _BPW * _L


def _posseg_body(pos_ref, seg_ref, out_ref):
    seg = seg_ref[...]
    pos = pos_ref[...]
    out_ref[...] = seg[:, None, :] + pos[None, :, :]


def _posseg(pos_table, seg_table):
    out = pl.pallas_call(
        _posseg_body,
        out_shape=jax.ShapeDtypeStruct((3, _L, _EMB), jnp.float32),
    )(pos_table, seg_table)
    return out.reshape(3 * _L, _EMB)


def _sc_body(x_hbm, s_hbm, tt_hbm, ps_hbm, out_hbm,
             idx_v, psidx_v, buf_v, tok_sems, add_sems, wr_sems, seg_sem):
    w = lax.axis_index("s") * _NC + lax.axis_index("c")
    b0 = w * _BPW

    pltpu.sync_copy(x_hbm.at[pl.ds(b0, _BPW)], idx_v)
    for r in range(_BPW):
        pltpu.async_copy(s_hbm.at[b0 + r], psidx_v.at[pl.ds(r * _L, _L)],
                         seg_sem)
    for r in range(_BPW):
        pltpu.make_async_copy(s_hbm.at[b0 + r],
                              psidx_v.at[pl.ds(r * _L, _L)], seg_sem).wait()

    iota = lax.iota(jnp.int32, 16)

    def idx_body(g, carry):
        f = g * 16
        s16 = psidx_v[pl.ds(f, 16)]
        psidx_v[pl.ds(f, 16)] = s16 * _L + lax.rem(f + iota, _L)
        return carry

    lax.fori_loop(0, _TPW // 16, idx_body, 0)

    def row_body(r, carry):
        b = b0 + r
        for h in range(_NH):
            @pl.when(r > 0)
            def _():
                pltpu.make_async_copy(
                    buf_v.at[h],
                    out_hbm.at[b - 1, pl.ds(h * _CH, _CH), pl.ds(0, _EMB)],
                    wr_sems.at[h]).wait()

            pltpu.async_copy(tt_hbm.at[idx_v.at[r, pl.ds(h * _CH, _CH)]],
                             buf_v.at[h], tok_sems.at[h])
        for h in range(_NH):
            pltpu.make_async_copy(
                tt_hbm.at[idx_v.at[r, pl.ds(h * _CH, _CH)]], buf_v.at[h],
                tok_sems.at[h]).wait()
            pltpu.async_copy(ps_hbm.at[psidx_v.at[pl.ds(r * _L + h * _CH, _CH)]],
                             buf_v.at[h], add_sems.at[h], add=True)
        for h in range(_NH):
            pltpu.make_async_copy(
                ps_hbm.at[psidx_v.at[pl.ds(r * _L + h * _CH, _CH)]], buf_v.at[h],
                add_sems.at[h]).wait()
            pltpu.async_copy(buf_v.at[h],
                             out_hbm.at[b, pl.ds(h * _CH, _CH), pl.ds(0, _EMB)],
                             wr_sems.at[h])
        return carry

    lax.fori_loop(0, _BPW, row_body, 0)

    for h in range(_NH):
        pltpu.make_async_copy(
            buf_v.at[h],
            out_hbm.at[b0 + _BPW - 1, pl.ds(h * _CH, _CH), pl.ds(0, _EMB)],
            wr_sems.at[h]).wait()


def _sc_call(x, segment_label, token_table, posseg):
    mesh = plsc.VectorSubcoreMesh(core_axis_name="c", subcore_axis_name="s")
    fn = pl.kernel(
        _sc_body,
        out_type=jax.ShapeDtypeStruct((_B, _L, 2 * _EMB), jnp.float32),
        mesh=mesh,
        compiler_params=pltpu.CompilerParams(use_tc_tiling_on_sc=False),
        scratch_types=[
            pltpu.VMEM((_BPW, _L), jnp.int32),
            pltpu.VMEM((_TPW,), jnp.int32),
            pltpu.VMEM((_NH, _CH, _EMB), jnp.float32),
            pltpu.SemaphoreType.DMA((_NH,)),
            pltpu.SemaphoreType.DMA((_NH,)),
            pltpu.SemaphoreType.DMA((_NH,)),
            pltpu.SemaphoreType.DMA,
        ],
    )
    return fn(x, segment_label, token_table, posseg)


def kernel(x, segment_label, token_table, pos_table, seg_table):
    posseg = _posseg(pos_table, seg_table)
    tt_c = jex_layout.with_layout_constraint(
        token_table,
        jex_layout.Layout(major_to_minor=(0, 1), tiling=((8,), (1024,))))
    out = _sc_call(x.astype(jnp.int32), segment_label.astype(jnp.int32),
                    tt_c, posseg)
    return out[:, :, :_EMB]
